# baseline (speedup 1.0000x reference)
import functools

import jax
import jax.numpy as jnp
from jax import lax
from jax.experimental import pallas as pl
from jax.experimental.pallas import tpu as pltpu
from jax.experimental.pallas import tpu_sc as plsc

_NC, _NS = 2, 16
_NW = _NC * _NS
_CHUNK = 4096


def _sc_counts(x_flat, t_flat, B, C, N):
    per_w = N // _NW
    G = per_w // _CHUNK
    mesh = plsc.VectorSubcoreMesh(core_axis_name="c", subcore_axis_name="s")

    @functools.partial(
        pl.kernel, mesh=mesh,
        out_type=jax.ShapeDtypeStruct((_NW * B * 3 * 16,), jnp.float32),
        scratch_types=[
            pltpu.VMEM((C, _CHUNK), jnp.float32),
            pltpu.VMEM((_CHUNK,), jnp.int32),
        ] + [pltpu.VMEM((16,), jnp.float32) for _ in range(6)],
        compiler_params=pltpu.CompilerParams(needs_layout_passes=False),
    )
    def k(x_hbm, t_hbm, out_hbm, xbuf, tbuf, *accs):
        wid = lax.axis_index("s") * _NC + lax.axis_index("c")
        base = wid * per_w
        zero16 = jnp.zeros((16,), jnp.float32)
        ones = jnp.ones((16,), jnp.float32)
        for a in accs:
            a[...] = zero16
        for b in range(B):
            acc_tp = accs[3 * b + 0]
            acc_cp = accs[3 * b + 1]
            acc_ct = accs[3 * b + 2]

            def chunk_body(g, carry, b=b, acc_tp=acc_tp, acc_cp=acc_cp,
                           acc_ct=acc_ct):
                off = base + g * _CHUNK
                for c in range(C):
                    pltpu.sync_copy(
                        x_hbm.at[pl.ds((b * C + c) * N + off, _CHUNK)],
                        xbuf.at[c])
                pltpu.sync_copy(t_hbm.at[pl.ds(b * N + off, _CHUNK)], tbuf)

                def vec_body(j, carry2):
                    s = pl.ds(j * 16, 16)
                    best = xbuf[0, s]
                    pred = jnp.zeros((16,), jnp.int32)
                    for c in range(1, C):
                        xc = xbuf[c, s]
                        m = xc > best
                        best = jnp.where(m, xc, best)
                        pred = jnp.where(m, c, pred)
                    tg = tbuf[s]
                    eq = pred == tg
                    plsc.addupdate_scatter(acc_tp, [tg], ones, mask=eq)
                    plsc.addupdate_scatter(acc_cp, [pred], ones)
                    plsc.addupdate_scatter(acc_ct, [tg], ones)
                    return carry2

                return lax.fori_loop(0, _CHUNK // 16, vec_body, carry)

            lax.fori_loop(0, G, chunk_body, 0)
        for b in range(B):
            for r in range(3):
                off = ((wid * B + b) * 3 + r) * 16
                pltpu.sync_copy(accs[3 * b + r], out_hbm.at[pl.ds(off, 16)])

    return k(x_flat, t_flat)


def kernel(inputs, targets):
    eps = 1e-05
    B, C, D, H, W = inputs.shape
    N = D * H * W
    x_flat = inputs.reshape(-1)
    t_flat = targets.reshape(-1).astype(jnp.int32)
    parts = _sc_counts(x_flat, t_flat, B, C, N).reshape(_NW, B, 3, 16)
    cnt = parts.sum(axis=0)[:, :, :C]
    tp, cp, ct = cnt[:, 0], cnt[:, 1], cnt[:, 2]
    loss = 2.0 * tp / (cp + ct + eps)
    return loss[:, 1:].mean(axis=1)


# speedup vs baseline: 1.2725x; 1.2725x over previous
import functools

import jax
import jax.numpy as jnp
from jax import lax
from jax.experimental import pallas as pl
from jax.experimental.pallas import tpu as pltpu
from jax.experimental.pallas import tpu_sc as plsc

_NC, _NS = 2, 16
_NW = _NC * _NS
_CHUNK = 4096
_L = 512
_R = 256


def _sc_counts(x_flat, t_flat, B, C, N, N0, N_sc):
    per_w = N_sc // _NW
    G = per_w // _CHUNK
    mesh = plsc.VectorSubcoreMesh(core_axis_name="c", subcore_axis_name="s")

    @functools.partial(
        pl.kernel, mesh=mesh,
        out_type=jax.ShapeDtypeStruct((_NW * B * 3 * 16,), jnp.float32),
        scratch_types=[
            pltpu.VMEM((C, _CHUNK), jnp.float32),
            pltpu.VMEM((_CHUNK,), jnp.int32),
        ] + [pltpu.VMEM((16,), jnp.float32) for _ in range(6)],
        compiler_params=pltpu.CompilerParams(needs_layout_passes=False),
    )
    def k(x_hbm, t_hbm, out_hbm, xbuf, tbuf, *accs):
        wid = lax.axis_index("s") * _NC + lax.axis_index("c")
        base = N0 + wid * per_w
        zero16 = jnp.zeros((16,), jnp.float32)
        ones = jnp.ones((16,), jnp.float32)
        for a in accs:
            a[...] = zero16
        for b in range(B):
            acc_tp = accs[3 * b + 0]
            acc_cp = accs[3 * b + 1]
            acc_ct = accs[3 * b + 2]

            def chunk_body(g, carry, b=b, acc_tp=acc_tp, acc_cp=acc_cp,
                           acc_ct=acc_ct):
                off = base + g * _CHUNK
                for c in range(C):
                    pltpu.sync_copy(
                        x_hbm.at[pl.ds((b * C + c) * N + off, _CHUNK)],
                        xbuf.at[c])
                pltpu.sync_copy(t_hbm.at[pl.ds(b * N + off, _CHUNK)], tbuf)

                def vec_body(j, carry2):
                    s = pl.ds(j * 16, 16)
                    best = xbuf[0, s]
                    pred = jnp.zeros((16,), jnp.int32)
                    for c in range(1, C):
                        xc = xbuf[c, s]
                        m = xc > best
                        best = jnp.where(m, xc, best)
                        pred = jnp.where(m, c, pred)
                    tg = tbuf[s]
                    eq = pred == tg
                    plsc.addupdate_scatter(acc_tp, [tg], ones, mask=eq)
                    plsc.addupdate_scatter(acc_cp, [pred], ones)
                    plsc.addupdate_scatter(acc_ct, [tg], ones)
                    return carry2

                return lax.fori_loop(0, _CHUNK // 16, vec_body, carry)

            lax.fori_loop(0, G, chunk_body, 0)
        for b in range(B):
            for r in range(3):
                off = ((wid * B + b) * 3 + r) * 16
                pltpu.sync_copy(accs[3 * b + r], out_hbm.at[pl.ds(off, 16)])

    return k(x_flat, t_flat)


def _tc_body(x_ref, t_ref, o_ref):
    C = x_ref.shape[1]
    x = x_ref[0]
    tgt = t_ref[0]
    best = x[0]
    pred = jnp.zeros_like(tgt)
    for c in range(1, C):
        m = x[c] > best
        best = jnp.where(m, x[c], best)
        pred = jnp.where(m, c, pred)
    one = jnp.ones_like(best)
    zero = jnp.zeros_like(best)
    rows = []
    for c in range(C):
        pc = pred == c
        tc = tgt == c
        rows.append(jnp.sum(jnp.where(pc & tc, one, zero), axis=0, keepdims=True))
        rows.append(jnp.sum(jnp.where(pc, one, zero), axis=0, keepdims=True))
        rows.append(jnp.sum(jnp.where(tc, one, zero), axis=0, keepdims=True))
    cnt = jnp.concatenate(rows, axis=0)
    i = pl.program_id(1)

    @pl.when(i == 0)
    def _init():
        o_ref[0] = cnt

    @pl.when(i > 0)
    def _acc():
        o_ref[0] = o_ref[0] + cnt


def _tc_counts(x, t, B, C, S_tc):
    G = S_tc // _R
    counts = pl.pallas_call(
        _tc_body,
        grid=(B, G),
        in_specs=[
            pl.BlockSpec((1, C, _R, _L), lambda b, i: (b, 0, i, 0)),
            pl.BlockSpec((1, _R, _L), lambda b, i: (b, i, 0)),
        ],
        out_specs=pl.BlockSpec((1, 3 * C, _L), lambda b, i: (b, 0, 0)),
        out_shape=jax.ShapeDtypeStruct((B, 3 * C, _L), jnp.float32),
        compiler_params=pltpu.CompilerParams(
            dimension_semantics=("parallel", "arbitrary")),
    )(x, t)
    return counts.sum(axis=2).reshape(B, C, 3)


def kernel(inputs, targets):
    eps = 1e-05
    B, C, D, H, W = inputs.shape
    N = D * H * W
    S = N // _L
    S_sc = 1280
    S_tc = S - S_sc
    N0 = S_tc * _L
    N_sc = S_sc * _L

    x = inputs.reshape(B, C, S, _L)
    t = targets.reshape(B, S, _L).astype(jnp.int32)
    x_flat = inputs.reshape(-1)
    t_flat = t.reshape(-1)

    sc_parts = _sc_counts(x_flat, t_flat, B, C, N, N0, N_sc)
    tc_cnt = _tc_counts(x, t, B, C, S_tc)
    sc_cnt = sc_parts.reshape(_NW, B, 3, 16).sum(axis=0)[:, :, :C]

    tp = tc_cnt[..., 0] + sc_cnt[:, 0]
    cp = tc_cnt[..., 1] + sc_cnt[:, 1]
    ct = tc_cnt[..., 2] + sc_cnt[:, 2]
    loss = 2.0 * tp / (cp + ct + eps)
    return loss[:, 1:].mean(axis=1)


# speedup vs baseline: 5.3130x; 4.1751x over previous
import jax
import jax.numpy as jnp
from jax.experimental import pallas as pl
from jax.experimental.pallas import tpu as pltpu

_DD = 4


def _tc_body(x_ref, t_ref, o_ref):
    C = x_ref.shape[1]
    W = x_ref.shape[4]
    x = x_ref[0]
    tgt = t_ref[0]
    best = x[0]
    pred = jnp.zeros_like(tgt)
    for c in range(1, C):
        m = x[c] > best
        best = jnp.where(m, x[c], best)
        pred = jnp.where(m, c, pred)
    one = jnp.ones_like(best)
    zero = jnp.zeros_like(best)
    rows = []
    for c in range(C):
        pc = pred == c
        tc = tgt == c
        for msk in (pc & tc, pc, tc):
            r = jnp.sum(jnp.where(msk, one, zero), axis=(0, 1), keepdims=True)
            rows.append(r.reshape(1, W))
    cnt = jnp.concatenate(rows, axis=0)
    i = pl.program_id(1)

    @pl.when(i == 0)
    def _init():
        o_ref[0] = cnt

    @pl.when(i > 0)
    def _acc():
        o_ref[0] = o_ref[0] + cnt


def kernel(inputs, targets):
    eps = 1e-05
    B, C, D, H, W = inputs.shape
    t = targets.astype(jnp.int32)
    G = D // _DD
    counts = pl.pallas_call(
        _tc_body,
        grid=(B, G),
        in_specs=[
            pl.BlockSpec((1, C, _DD, H, W), lambda b, i: (b, 0, i, 0, 0)),
            pl.BlockSpec((1, _DD, H, W), lambda b, i: (b, i, 0, 0)),
        ],
        out_specs=pl.BlockSpec((1, 3 * C, W), lambda b, i: (b, 0, 0)),
        out_shape=jax.ShapeDtypeStruct((B, 3 * C, W), jnp.float32),
        compiler_params=pltpu.CompilerParams(
            dimension_semantics=("parallel", "arbitrary")),
    )(inputs, t)
    cnt = counts.sum(axis=2).reshape(B, C, 3)
    tp, cp, ct = cnt[..., 0], cnt[..., 1], cnt[..., 2]
    loss = 2.0 * tp / (cp + ct + eps)
    return loss[:, 1:].mean(axis=1)
